# baseline (speedup 1.0000x reference)
import functools

import jax
import jax.numpy as jnp
from jax import lax
from jax.experimental import pallas as pl
from jax.experimental.pallas import tpu as pltpu
from jax.experimental.pallas import tpu_sc as plsc

_HID = 256
_N = 10000
_E = 160000
_HALF = 128
_CW = 128
_DR = 80
_NSUB = 16
_EPT = _E // _NSUB
_C = 80
_SEG = 5
_ECS = _EPT // _SEG
_NCSEG = _ECS // _C
_NA = 10240
_RPT = _NA // _NSUB
_BLK = 400
_NBLK = _N // _BLK


@functools.cache
def _sc_agg_fn():
    mesh = plsc.VectorSubcoreMesh(core_axis_name="c", subcore_axis_name="s")
    return pl.kernel(
        _sc_body,
        out_type=[jax.ShapeDtypeStruct((2 * _NA, _CW), jnp.float32),
                  jax.ShapeDtypeStruct((2 * _NA, _CW), jnp.float32),
                  jax.ShapeDtypeStruct((2 * _DR, 128), jnp.float32),
                  jax.ShapeDtypeStruct((2 * _DR, 128), jnp.float32)],
        mesh=mesh,
        scratch_types=[
            pltpu.VMEM((_ECS,), jnp.int32),
            pltpu.VMEM((_ECS,), jnp.int32),
            pltpu.VMEM((_C,), jnp.int32),
            pltpu.VMEM((_C,), jnp.int32),
            pltpu.VMEM((_C,), jnp.int32),
            pltpu.VMEM((_C,), jnp.int32),
            pltpu.VMEM((_C,), jnp.int32),
            pltpu.VMEM((_C,), jnp.int32),
            pltpu.VMEM((_DR,), jnp.int32),
            pltpu.VMEM((_C, _CW), jnp.float32),
            pltpu.VMEM((_C, _CW), jnp.float32),
            pltpu.VMEM((_C, _CW), jnp.float32),
            pltpu.VMEM((_DR, 128), jnp.float32),
            pltpu.VMEM_SHARED((_NA, _CW), jnp.float32),
            pltpu.VMEM_SHARED((_DR, 128), jnp.float32),
            pltpu.SemaphoreType.DMA,
            pltpu.SemaphoreType.DMA,
            pltpu.SemaphoreType.DMA,
            pltpu.SemaphoreType.DMA,
            pltpu.SemaphoreType.DMA,
            pltpu.SemaphoreType.DMA,
        ],
        compiler_params=pltpu.CompilerParams(needs_layout_passes=False),
    )


def _sc_body(tu, ti, src_ui, dst_ui, src_iu, dst_iu, zeros,
             out_i, out_u, out_degi, out_degu,
             sall, dall, idx_s0, idx_d0, idx_s1, idx_d1, idx_s2, idx_d2,
             idx_io, rows0, rows1, rows2, degv, acc, deg_acc,
             gsem0, gsem1, gsem2, ssem0, ssem1, ssem2):
    cid = lax.axis_index("c")
    sid = lax.axis_index("s")
    colo = pl.multiple_of(cid * _HALF, _HALF)
    offa = cid * _NA

    for j in range(_DR // 16):
        idx_io[pl.ds(j * 16, 16)] = lax.iota(jnp.int32, 16) + (j * 16)
    ones16 = jnp.full((16,), 1.0, jnp.float32)

    def run_dir(table, src, dst, out, out_deg):
        pltpu.sync_copy(zeros, acc.at[pl.ds(sid * _RPT, _RPT)])
        pltpu.sync_copy(zeros.at[pl.ds(0, _DR)], degv)

        @pl.when(sid == 0)
        def _():
            pltpu.sync_copy(zeros.at[pl.ds(0, _DR)], deg_acc)

        plsc.subcore_barrier()

        pairs = ((idx_s0, idx_d0, rows0, gsem0, ssem0),
                 (idx_s1, idx_d1, rows1, gsem1, ssem1),
                 (idx_s2, idx_d2, rows2, gsem2, ssem2))

        def fire(kf, isb, idb, rb, gsem, ssem):
            @pl.when(kf >= 3)
            def _():
                pltpu.make_async_copy(rb, acc.at[idb], ssem).wait()

            base = kf * _C
            for j in range(_C // 16):
                sl = pl.ds(j * 16, 16)
                isb[sl] = sall[pl.ds(base + j * 16, 16)]
                idb[sl] = dall[pl.ds(base + j * 16, 16)]
            pltpu.async_copy(table.at[isb, pl.ds(colo, _HALF)], rb, gsem)

        def consume(isb, idb, rb, gsem, ssem):
            pltpu.make_async_copy(table.at[isb, pl.ds(colo, _HALF)],
                                  rb, gsem).wait()
            pltpu.async_copy(rb, acc.at[idb], ssem, add=True)
            for j in range(_C // 16):
                d16 = idb[pl.ds(j * 16, 16)]
                plsc.addupdate_scatter(
                    degv, [lax.shift_right_logical(d16, 7),
                           lax.bitwise_and(d16, 127)], ones16)

        def segment(s, carry0):
            pltpu.sync_copy(src.at[pl.ds(sid * _EPT + s * _ECS, _ECS)], sall)
            pltpu.sync_copy(dst.at[pl.ds(sid * _EPT + s * _ECS, _ECS)], dall)
            fire(0, *pairs[0])
            fire(1, *pairs[1])

            def chunk(k, carry):
                kf = k + 2
                for p in range(3):
                    @pl.when(jnp.logical_and(kf < _NCSEG,
                                             lax.rem(kf, 3) == p))
                    def _(p=p):
                        fire(kf, *pairs[p])

                for p in range(3):
                    @pl.when(lax.rem(k, 3) == p)
                    def _(p=p):
                        consume(*pairs[p])

                return carry

            lax.fori_loop(0, _NCSEG, chunk, 0)
            for p in range(3):
                isb, idb, rb, gsem, ssem = pairs[p]
                pltpu.make_async_copy(rb, acc.at[idb], ssem).wait()
            return carry0

        lax.fori_loop(0, _SEG, segment, 0)
        pltpu.sync_copy(degv, deg_acc.at[idx_io], add=True)
        plsc.subcore_barrier()
        pltpu.sync_copy(acc.at[pl.ds(sid * _RPT, _RPT)],
                        out.at[pl.ds(offa + sid * _RPT, _RPT)])

        @pl.when(sid == 0)
        def _():
            pltpu.sync_copy(deg_acc, out_deg.at[pl.ds(cid * _DR, _DR)])

        plsc.subcore_barrier()

    run_dir(tu, src_ui, dst_ui, out_i, out_degi)
    run_dir(ti, src_iu, dst_iu, out_u, out_degu)


def _tc_z_body(xu, xi, wu, wi, zu, zi):
    zu[...] = jnp.dot(xu[...], wu[...], preferred_element_type=jnp.float32)
    zi[...] = jnp.dot(xi[...], wi[...], preferred_element_type=jnp.float32)


def _self_pass(xu, xi, wu, wi):
    blk = lambda: pl.BlockSpec((_BLK, _HID), lambda i: (i, 0))
    wspec = lambda: pl.BlockSpec((_HID, _HID), lambda i: (0, 0))
    return pl.pallas_call(
        _tc_z_body,
        grid=(_NBLK,),
        in_specs=[blk(), blk(), wspec(), wspec()],
        out_specs=[blk(), blk()],
        out_shape=[jax.ShapeDtypeStruct((_N, _HID), jnp.float32),
                   jax.ShapeDtypeStruct((_N, _HID), jnp.float32)],
    )(xu, xi, wu, wi)


def _tc_a_body(zu, zi, au0, au1, ai0, ai1, degu, degi, wiu, wui,
               yu, yi, stats):
    i = pl.program_id(0)
    aggu = jnp.concatenate([au0[0], au1[0]], axis=1)
    aggi = jnp.concatenate([ai0[0], ai1[0]], axis=1)
    du = jnp.maximum(degu[...], 1.0)
    di = jnp.maximum(degi[...], 1.0)
    mu = jnp.dot(aggu, wiu[...], preferred_element_type=jnp.float32) / du
    yu_v = jnp.maximum(mu + zu[...], 0.0)
    mi = jnp.dot(aggi, wui[...], preferred_element_type=jnp.float32) / di
    yi_v = jnp.maximum(mi + zi[...], 0.0)
    yu[...] = yu_v
    yi[...] = yi_v
    s = jnp.concatenate([
        jnp.sum(yu_v, axis=0, keepdims=True),
        jnp.sum(yu_v * yu_v, axis=0, keepdims=True),
        jnp.sum(yi_v, axis=0, keepdims=True),
        jnp.sum(yi_v * yi_v, axis=0, keepdims=True),
    ], axis=0)

    @pl.when(i == 0)
    def _():
        stats[...] = s

    @pl.when(i != 0)
    def _():
        stats[...] = stats[...] + s


def _dense_pass(zu, zi, aggu3, aggi3, degu, degi, wiu, wui):
    blk = lambda: pl.BlockSpec((_BLK, _HID), lambda i: (i, 0))
    h0 = lambda: pl.BlockSpec((1, _BLK, _HALF), lambda i: (0, i, 0))
    h1 = lambda: pl.BlockSpec((1, _BLK, _HALF), lambda i: (1, i, 0))
    wspec = lambda: pl.BlockSpec((_HID, _HID), lambda i: (0, 0))
    return pl.pallas_call(
        _tc_a_body,
        grid=(_NBLK,),
        in_specs=[blk(), blk(), h0(), h1(), h0(), h1(),
                  pl.BlockSpec((_BLK, 1), lambda i: (i, 0)),
                  pl.BlockSpec((_BLK, 1), lambda i: (i, 0)),
                  wspec(), wspec()],
        out_specs=[blk(), blk(),
                   pl.BlockSpec((4, _HID), lambda i: (0, 0))],
        out_shape=[jax.ShapeDtypeStruct((_N, _HID), jnp.float32),
                   jax.ShapeDtypeStruct((_N, _HID), jnp.float32),
                   jax.ShapeDtypeStruct((4, _HID), jnp.float32)],
    )(zu, zi, aggu3, aggu3, aggi3, aggi3, degu, degi, wiu, wui)


def _tc_b_body(yu, yi, stats, g, b, lp, ou, oi):
    n = float(_N)
    lpv = lp[...]
    gv = g[...]
    bv = b[...]

    def norm(y, r):
        mean = stats[r:r + 1, :] / n
        var = stats[r + 1:r + 2, :] / n - mean * mean
        grstd = gv * lax.rsqrt(var + 1e-5)
        a = lpv * grstd + (1.0 - lpv)
        c = lpv * (bv - mean * grstd)
        return y[...] * a + c

    ou[...] = norm(yu, 0)
    oi[...] = norm(yi, 2)


def _bn_pass(yu, yi, stats, g, b, lp):
    blk = lambda: pl.BlockSpec((_BLK, _HID), lambda i: (i, 0))
    row = lambda r: pl.BlockSpec((r, _HID), lambda i: (0, 0))
    return pl.pallas_call(
        _tc_b_body,
        grid=(_NBLK,),
        in_specs=[blk(), blk(), row(4), row(1), row(1), row(1)],
        out_specs=[blk(), blk()],
        out_shape=[jax.ShapeDtypeStruct((_N, _HID), jnp.float32),
                   jax.ShapeDtypeStruct((_N, _HID), jnp.float32)],
    )(yu, yi, stats, g, b, lp)


def kernel(x_user, x_item, edge_index_ui, edge_index_iu, LP,
           W_ui_0, W_iu_0, Wu_0, Wi_0, g_0, b_0,
           W_ui_1, W_iu_1, Wu_1, Wi_1, g_1, b_1):
    f32 = jnp.float32
    xu = x_user.astype(f32)
    xi = x_item.astype(f32)
    ei_ui = edge_index_ui.astype(jnp.int32)
    ei_iu = edge_index_iu.astype(jnp.int32)

    zeros = jnp.zeros((_RPT, _CW), f32)

    out_i, out_u, out_degi, out_degu = _sc_agg_fn()(
        xu, xi, ei_ui[0], ei_ui[1], ei_iu[0], ei_iu[1], zeros)

    zu, zi = _self_pass(xu, xi, Wu_1.astype(f32), Wi_1.astype(f32))

    aggi3 = out_i.reshape(2, _NA, _CW)
    aggu3 = out_u.reshape(2, _NA, _CW)
    degi = out_degi[:_DR].reshape(_NA)[:_N, None]
    degu = out_degu[:_DR].reshape(_NA)[:_N, None]

    yu, yi, stats = _dense_pass(zu, zi, aggu3, aggi3, degu, degi,
                                W_iu_1.astype(f32), W_ui_1.astype(f32))

    lp = jnp.where(LP != 0, 1.0, 0.0).astype(f32) * jnp.ones((1, _HID), f32)
    ou, oi = _bn_pass(yu, yi, stats,
                      g_1.reshape(1, _HID).astype(f32),
                      b_1.reshape(1, _HID).astype(f32), lp)
    return ou[None, :, :], oi[None, :, :]


# speedup vs baseline: 1.0156x; 1.0156x over previous
import functools

import jax
import jax.numpy as jnp
from jax import lax
from jax.experimental import pallas as pl
from jax.experimental.pallas import tpu as pltpu
from jax.experimental.pallas import tpu_sc as plsc

_HID = 256
_N = 10000
_E = 160000
_HALF = 128
_CW = 128
_DR = 80
_NSUB = 16
_EPT = _E // _NSUB
_C = 80
_SEG = 5
_ECS = _EPT // _SEG
_NCSEG = _ECS // _C
_NA = 10240
_RPT = _NA // _NSUB
_BLK = 400
_NBLK = _N // _BLK


@functools.cache
def _sc_agg_fn():
    mesh = plsc.VectorSubcoreMesh(core_axis_name="c", subcore_axis_name="s")
    return pl.kernel(
        _sc_body,
        out_type=[jax.ShapeDtypeStruct((2 * _NA, _CW), jnp.float32),
                  jax.ShapeDtypeStruct((2 * _NA, _CW), jnp.float32),
                  jax.ShapeDtypeStruct((2 * _DR, 128), jnp.float32),
                  jax.ShapeDtypeStruct((2 * _DR, 128), jnp.float32)],
        mesh=mesh,
        scratch_types=[
            pltpu.VMEM((_ECS,), jnp.int32),
            pltpu.VMEM((_ECS,), jnp.int32),
            pltpu.VMEM((_C,), jnp.int32),
            pltpu.VMEM((_C,), jnp.int32),
            pltpu.VMEM((_C,), jnp.int32),
            pltpu.VMEM((_C,), jnp.int32),
            pltpu.VMEM((_C,), jnp.int32),
            pltpu.VMEM((_C,), jnp.int32),
            pltpu.VMEM((_DR,), jnp.int32),
            pltpu.VMEM((_C, _CW), jnp.float32),
            pltpu.VMEM((_C, _CW), jnp.float32),
            pltpu.VMEM((_C, _CW), jnp.float32),
            pltpu.VMEM((_DR, 128), jnp.float32),
            pltpu.VMEM_SHARED((_NA, _CW), jnp.float32),
            pltpu.VMEM_SHARED((_DR, 128), jnp.float32),
            pltpu.SemaphoreType.DMA,
            pltpu.SemaphoreType.DMA,
            pltpu.SemaphoreType.DMA,
            pltpu.SemaphoreType.DMA,
            pltpu.SemaphoreType.DMA,
            pltpu.SemaphoreType.DMA,
        ],
        compiler_params=pltpu.CompilerParams(needs_layout_passes=False),
    )


def _sc_body(tu, ti, src_ui, dst_ui, src_iu, dst_iu, zeros,
             out_i, out_u, out_degi, out_degu,
             sall, dall, idx_s0, idx_d0, idx_s1, idx_d1, idx_s2, idx_d2,
             idx_io, rows0, rows1, rows2, degv, acc, deg_acc,
             gsem0, gsem1, gsem2, ssem0, ssem1, ssem2):
    cid = lax.axis_index("c")
    sid = lax.axis_index("s")
    colo = pl.multiple_of(cid * _HALF, _HALF)
    offa = cid * _NA

    for j in range(_DR // 16):
        idx_io[pl.ds(j * 16, 16)] = lax.iota(jnp.int32, 16) + (j * 16)
    ones16 = jnp.full((16,), 1.0, jnp.float32)

    def run_dir(table, src, dst, out, out_deg):
        pltpu.sync_copy(zeros, acc.at[pl.ds(sid * _RPT, _RPT)])
        pltpu.sync_copy(zeros.at[pl.ds(0, _DR)], degv)

        @pl.when(sid == 0)
        def _():
            pltpu.sync_copy(zeros.at[pl.ds(0, _DR)], deg_acc)

        plsc.subcore_barrier()

        pairs = ((idx_s0, idx_d0, rows0, gsem0, ssem0),
                 (idx_s1, idx_d1, rows1, gsem1, ssem1),
                 (idx_s2, idx_d2, rows2, gsem2, ssem2))

        def fire(kf, isb, idb, rb, gsem, ssem):
            @pl.when(kf >= 3)
            def _():
                pltpu.make_async_copy(rb, acc.at[idb], ssem).wait()

            base = kf * _C
            for j in range(_C // 16):
                sl = pl.ds(j * 16, 16)
                isb[sl] = sall[pl.ds(base + j * 16, 16)]
                idb[sl] = dall[pl.ds(base + j * 16, 16)]
            pltpu.async_copy(table.at[isb, pl.ds(colo, _HALF)], rb, gsem)

        def consume(isb, idb, rb, gsem, ssem):
            pltpu.make_async_copy(table.at[isb, pl.ds(colo, _HALF)],
                                  rb, gsem).wait()
            pltpu.async_copy(rb, acc.at[idb], ssem, add=True)
            for j in range(_C // 16):
                d16 = idb[pl.ds(j * 16, 16)]
                plsc.addupdate_scatter(
                    degv, [lax.shift_right_logical(d16, 7),
                           lax.bitwise_and(d16, 127)], ones16)

        def segment(s, carry0):
            pltpu.sync_copy(src.at[pl.ds(sid * _EPT + s * _ECS, _ECS)], sall)
            pltpu.sync_copy(dst.at[pl.ds(sid * _EPT + s * _ECS, _ECS)], dall)
            fire(0, *pairs[0])
            fire(1, *pairs[1])

            def chunk(k, carry):
                kf = k + 2
                for p in range(3):
                    @pl.when(jnp.logical_and(kf < _NCSEG,
                                             lax.rem(kf, 3) == p))
                    def _(p=p):
                        fire(kf, *pairs[p])

                for p in range(3):
                    @pl.when(lax.rem(k, 3) == p)
                    def _(p=p):
                        consume(*pairs[p])

                return carry

            lax.fori_loop(0, _NCSEG, chunk, 0)
            for p in range(3):
                isb, idb, rb, gsem, ssem = pairs[p]
                pltpu.make_async_copy(rb, acc.at[idb], ssem).wait()
            return carry0

        lax.fori_loop(0, _SEG, segment, 0)
        pltpu.sync_copy(degv, deg_acc.at[idx_io], add=True)
        plsc.subcore_barrier()
        pltpu.sync_copy(acc.at[pl.ds(sid * _RPT, _RPT)],
                        out.at[pl.ds(offa + sid * _RPT, _RPT)])

        @pl.when(sid == 0)
        def _():
            pltpu.sync_copy(deg_acc, out_deg.at[pl.ds(cid * _DR, _DR)])

        plsc.subcore_barrier()

    run_dir(tu, src_ui, dst_ui, out_i, out_degi)
    run_dir(ti, src_iu, dst_iu, out_u, out_degu)


def _tc_a_body(xu, xi, au0, au1, ai0, ai1, degu, degi, wiu, wu, wui, wi,
               yu, yi, stats):
    i = pl.program_id(0)
    aggu = jnp.concatenate([au0[0], au1[0]], axis=1)
    aggi = jnp.concatenate([ai0[0], ai1[0]], axis=1)
    du = jnp.maximum(degu[...], 1.0)
    di = jnp.maximum(degi[...], 1.0)
    mu = jnp.dot(aggu, wiu[...], preferred_element_type=jnp.float32) / du
    yu_v = jnp.maximum(
        mu + jnp.dot(xu[...], wu[...], preferred_element_type=jnp.float32), 0.0)
    mi = jnp.dot(aggi, wui[...], preferred_element_type=jnp.float32) / di
    yi_v = jnp.maximum(
        mi + jnp.dot(xi[...], wi[...], preferred_element_type=jnp.float32), 0.0)
    yu[...] = yu_v
    yi[...] = yi_v
    s = jnp.concatenate([
        jnp.sum(yu_v, axis=0, keepdims=True),
        jnp.sum(yu_v * yu_v, axis=0, keepdims=True),
        jnp.sum(yi_v, axis=0, keepdims=True),
        jnp.sum(yi_v * yi_v, axis=0, keepdims=True),
    ], axis=0)

    @pl.when(i == 0)
    def _():
        stats[...] = s

    @pl.when(i != 0)
    def _():
        stats[...] = stats[...] + s


def _dense_pass(xu, xi, aggu3, aggi3, degu, degi, wiu, wu, wui, wi):
    blk = lambda: pl.BlockSpec((_BLK, _HID), lambda i: (i, 0))
    h0 = lambda: pl.BlockSpec((1, _BLK, _HALF), lambda i: (0, i, 0))
    h1 = lambda: pl.BlockSpec((1, _BLK, _HALF), lambda i: (1, i, 0))
    wspec = lambda: pl.BlockSpec((_HID, _HID), lambda i: (0, 0))
    return pl.pallas_call(
        _tc_a_body,
        grid=(_NBLK,),
        in_specs=[blk(), blk(), h0(), h1(), h0(), h1(),
                  pl.BlockSpec((_BLK, 1), lambda i: (i, 0)),
                  pl.BlockSpec((_BLK, 1), lambda i: (i, 0)),
                  wspec(), wspec(), wspec(), wspec()],
        out_specs=[blk(), blk(),
                   pl.BlockSpec((4, _HID), lambda i: (0, 0))],
        out_shape=[jax.ShapeDtypeStruct((_N, _HID), jnp.float32),
                   jax.ShapeDtypeStruct((_N, _HID), jnp.float32),
                   jax.ShapeDtypeStruct((4, _HID), jnp.float32)],
    )(xu, xi, aggu3, aggu3, aggi3, aggi3, degu, degi, wiu, wu, wui, wi)


def _tc_b_body(yu, yi, stats, g, b, lp, ou, oi):
    n = float(_N)
    lpv = lp[...]
    gv = g[...]
    bv = b[...]

    def norm(y, r):
        mean = stats[r:r + 1, :] / n
        var = stats[r + 1:r + 2, :] / n - mean * mean
        grstd = gv * lax.rsqrt(var + 1e-5)
        a = lpv * grstd + (1.0 - lpv)
        c = lpv * (bv - mean * grstd)
        return y[...] * a + c

    ou[...] = norm(yu, 0)
    oi[...] = norm(yi, 2)


def _bn_pass(yu, yi, stats, g, b, lp):
    blk = lambda: pl.BlockSpec((_BLK, _HID), lambda i: (i, 0))
    row = lambda r: pl.BlockSpec((r, _HID), lambda i: (0, 0))
    return pl.pallas_call(
        _tc_b_body,
        grid=(_NBLK,),
        in_specs=[blk(), blk(), row(4), row(1), row(1), row(1)],
        out_specs=[blk(), blk()],
        out_shape=[jax.ShapeDtypeStruct((_N, _HID), jnp.float32),
                   jax.ShapeDtypeStruct((_N, _HID), jnp.float32)],
    )(yu, yi, stats, g, b, lp)


def kernel(x_user, x_item, edge_index_ui, edge_index_iu, LP,
           W_ui_0, W_iu_0, Wu_0, Wi_0, g_0, b_0,
           W_ui_1, W_iu_1, Wu_1, Wi_1, g_1, b_1):
    f32 = jnp.float32
    xu = x_user.astype(f32)
    xi = x_item.astype(f32)
    ei_ui = edge_index_ui.astype(jnp.int32)
    ei_iu = edge_index_iu.astype(jnp.int32)

    zeros = jnp.zeros((_RPT, _CW), f32)

    out_i, out_u, out_degi, out_degu = _sc_agg_fn()(
        xu, xi, ei_ui[0], ei_ui[1], ei_iu[0], ei_iu[1], zeros)

    aggi3 = out_i.reshape(2, _NA, _CW)
    aggu3 = out_u.reshape(2, _NA, _CW)
    degi = out_degi[:_DR].reshape(_NA)[:_N, None]
    degu = out_degu[:_DR].reshape(_NA)[:_N, None]

    yu, yi, stats = _dense_pass(xu, xi, aggu3, aggi3, degu, degi,
                                W_iu_1.astype(f32), Wu_1.astype(f32),
                                W_ui_1.astype(f32), Wi_1.astype(f32))

    lp = jnp.where(LP != 0, 1.0, 0.0).astype(f32) * jnp.ones((1, _HID), f32)
    ou, oi = _bn_pass(yu, yi, stats,
                      g_1.reshape(1, _HID).astype(f32),
                      b_1.reshape(1, _HID).astype(f32), lp)
    return ou[None, :, :], oi[None, :, :]


# speedup vs baseline: 1.0653x; 1.0489x over previous
import functools

import jax
import jax.numpy as jnp
from jax import lax
from jax.experimental import pallas as pl
from jax.experimental.pallas import tpu as pltpu
from jax.experimental.pallas import tpu_sc as plsc

_HID = 256
_N = 10000
_E = 160000
_HALF = 128
_CW = 128
_DR = 80
_NSUB = 16
_EPT = _E // _NSUB
_C = 80
_SEG = 5
_ECS = _EPT // _SEG
_NCSEG = _ECS // _C
_NA = 10240
_RPT = _NA // _NSUB
_BLK = 1000
_NBLK = _N // _BLK


@functools.cache
def _sc_agg_fn():
    mesh = plsc.VectorSubcoreMesh(core_axis_name="c", subcore_axis_name="s")
    return pl.kernel(
        _sc_body,
        out_type=[jax.ShapeDtypeStruct((2 * _NA, _CW), jnp.float32),
                  jax.ShapeDtypeStruct((2 * _NA, _CW), jnp.float32),
                  jax.ShapeDtypeStruct((2 * _DR, 128), jnp.float32),
                  jax.ShapeDtypeStruct((2 * _DR, 128), jnp.float32)],
        mesh=mesh,
        scratch_types=[
            pltpu.VMEM((_ECS,), jnp.int32),
            pltpu.VMEM((_ECS,), jnp.int32),
            pltpu.VMEM((_C,), jnp.int32),
            pltpu.VMEM((_C,), jnp.int32),
            pltpu.VMEM((_C,), jnp.int32),
            pltpu.VMEM((_C,), jnp.int32),
            pltpu.VMEM((_C,), jnp.int32),
            pltpu.VMEM((_C,), jnp.int32),
            pltpu.VMEM((_DR,), jnp.int32),
            pltpu.VMEM((_C, _CW), jnp.float32),
            pltpu.VMEM((_C, _CW), jnp.float32),
            pltpu.VMEM((_C, _CW), jnp.float32),
            pltpu.VMEM((_DR, 128), jnp.float32),
            pltpu.VMEM_SHARED((_NA, _CW), jnp.float32),
            pltpu.VMEM_SHARED((_DR, 128), jnp.float32),
            pltpu.SemaphoreType.DMA,
            pltpu.SemaphoreType.DMA,
            pltpu.SemaphoreType.DMA,
            pltpu.SemaphoreType.DMA,
            pltpu.SemaphoreType.DMA,
            pltpu.SemaphoreType.DMA,
        ],
        compiler_params=pltpu.CompilerParams(needs_layout_passes=False),
    )


def _sc_body(tu, ti, src_ui, dst_ui, src_iu, dst_iu, zeros,
             out_i, out_u, out_degi, out_degu,
             sall, dall, idx_s0, idx_d0, idx_s1, idx_d1, idx_s2, idx_d2,
             idx_io, rows0, rows1, rows2, degv, acc, deg_acc,
             gsem0, gsem1, gsem2, ssem0, ssem1, ssem2):
    cid = lax.axis_index("c")
    sid = lax.axis_index("s")
    colo = pl.multiple_of(cid * _HALF, _HALF)
    offa = cid * _NA

    for j in range(_DR // 16):
        idx_io[pl.ds(j * 16, 16)] = lax.iota(jnp.int32, 16) + (j * 16)
    ones16 = jnp.full((16,), 1.0, jnp.float32)

    def run_dir(table, src, dst, out, out_deg):
        pltpu.sync_copy(zeros, acc.at[pl.ds(sid * _RPT, _RPT)])
        pltpu.sync_copy(zeros.at[pl.ds(0, _DR)], degv)

        @pl.when(sid == 0)
        def _():
            pltpu.sync_copy(zeros.at[pl.ds(0, _DR)], deg_acc)

        plsc.subcore_barrier()

        pairs = ((idx_s0, idx_d0, rows0, gsem0, ssem0),
                 (idx_s1, idx_d1, rows1, gsem1, ssem1),
                 (idx_s2, idx_d2, rows2, gsem2, ssem2))

        def fire(kf, isb, idb, rb, gsem, ssem):
            @pl.when(kf >= 3)
            def _():
                pltpu.make_async_copy(rb, acc.at[idb], ssem).wait()

            base = kf * _C
            for j in range(_C // 16):
                sl = pl.ds(j * 16, 16)
                isb[sl] = sall[pl.ds(base + j * 16, 16)]
                idb[sl] = dall[pl.ds(base + j * 16, 16)]
            pltpu.async_copy(table.at[isb, pl.ds(colo, _HALF)], rb, gsem)

        def consume(isb, idb, rb, gsem, ssem):
            pltpu.make_async_copy(table.at[isb, pl.ds(colo, _HALF)],
                                  rb, gsem).wait()
            pltpu.async_copy(rb, acc.at[idb], ssem, add=True)
            for j in range(_C // 16):
                d16 = idb[pl.ds(j * 16, 16)]
                plsc.addupdate_scatter(
                    degv, [lax.shift_right_logical(d16, 7),
                           lax.bitwise_and(d16, 127)], ones16)

        def segment(s, carry0):
            pltpu.sync_copy(src.at[pl.ds(sid * _EPT + s * _ECS, _ECS)], sall)
            pltpu.sync_copy(dst.at[pl.ds(sid * _EPT + s * _ECS, _ECS)], dall)
            fire(0, *pairs[0])
            fire(1, *pairs[1])

            def chunk(k, carry):
                kf = k + 2
                for p in range(3):
                    @pl.when(jnp.logical_and(kf < _NCSEG,
                                             lax.rem(kf, 3) == p))
                    def _(p=p):
                        fire(kf, *pairs[p])

                for p in range(3):
                    @pl.when(lax.rem(k, 3) == p)
                    def _(p=p):
                        consume(*pairs[p])

                return carry

            lax.fori_loop(0, _NCSEG, chunk, 0)
            for p in range(3):
                isb, idb, rb, gsem, ssem = pairs[p]
                pltpu.make_async_copy(rb, acc.at[idb], ssem).wait()
            return carry0

        lax.fori_loop(0, _SEG, segment, 0)
        pltpu.sync_copy(degv, deg_acc.at[idx_io], add=True)
        plsc.subcore_barrier()
        pltpu.sync_copy(acc.at[pl.ds(sid * _RPT, _RPT)],
                        out.at[pl.ds(offa + sid * _RPT, _RPT)])

        @pl.when(sid == 0)
        def _():
            pltpu.sync_copy(deg_acc, out_deg.at[pl.ds(cid * _DR, _DR)])

        plsc.subcore_barrier()

    run_dir(tu, src_ui, dst_ui, out_i, out_degi)
    run_dir(ti, src_iu, dst_iu, out_u, out_degu)


def _tc_a_body(xu, xi, au0, au1, ai0, ai1, degu, degi, wiu, wu, wui, wi,
               yu, yi, stats):
    i = pl.program_id(0)
    aggu = jnp.concatenate([au0[0], au1[0]], axis=1)
    aggi = jnp.concatenate([ai0[0], ai1[0]], axis=1)
    du = jnp.maximum(degu[...], 1.0)
    di = jnp.maximum(degi[...], 1.0)
    mu = jnp.dot(aggu, wiu[...], preferred_element_type=jnp.float32) / du
    yu_v = jnp.maximum(
        mu + jnp.dot(xu[...], wu[...], preferred_element_type=jnp.float32), 0.0)
    mi = jnp.dot(aggi, wui[...], preferred_element_type=jnp.float32) / di
    yi_v = jnp.maximum(
        mi + jnp.dot(xi[...], wi[...], preferred_element_type=jnp.float32), 0.0)
    yu[...] = yu_v
    yi[...] = yi_v
    s = jnp.concatenate([
        jnp.sum(yu_v, axis=0, keepdims=True),
        jnp.sum(yu_v * yu_v, axis=0, keepdims=True),
        jnp.sum(yi_v, axis=0, keepdims=True),
        jnp.sum(yi_v * yi_v, axis=0, keepdims=True),
    ], axis=0)

    @pl.when(i == 0)
    def _():
        stats[...] = s

    @pl.when(i != 0)
    def _():
        stats[...] = stats[...] + s


def _dense_pass(xu, xi, aggu3, aggi3, degu, degi, wiu, wu, wui, wi):
    blk = lambda: pl.BlockSpec((_BLK, _HID), lambda i: (i, 0))
    h0 = lambda: pl.BlockSpec((1, _BLK, _HALF), lambda i: (0, i, 0))
    h1 = lambda: pl.BlockSpec((1, _BLK, _HALF), lambda i: (1, i, 0))
    wspec = lambda: pl.BlockSpec((_HID, _HID), lambda i: (0, 0))
    return pl.pallas_call(
        _tc_a_body,
        grid=(_NBLK,),
        in_specs=[blk(), blk(), h0(), h1(), h0(), h1(),
                  pl.BlockSpec((_BLK, 1), lambda i: (i, 0)),
                  pl.BlockSpec((_BLK, 1), lambda i: (i, 0)),
                  wspec(), wspec(), wspec(), wspec()],
        out_specs=[blk(), blk(),
                   pl.BlockSpec((4, _HID), lambda i: (0, 0))],
        out_shape=[jax.ShapeDtypeStruct((_N, _HID), jnp.float32),
                   jax.ShapeDtypeStruct((_N, _HID), jnp.float32),
                   jax.ShapeDtypeStruct((4, _HID), jnp.float32)],
    )(xu, xi, aggu3, aggu3, aggi3, aggi3, degu, degi, wiu, wu, wui, wi)


def _tc_b_body(yu, yi, stats, g, b, lp, ou, oi):
    n = float(_N)
    lpv = lp[...]
    gv = g[...]
    bv = b[...]

    def norm(y, r):
        mean = stats[r:r + 1, :] / n
        var = stats[r + 1:r + 2, :] / n - mean * mean
        grstd = gv * lax.rsqrt(var + 1e-5)
        a = lpv * grstd + (1.0 - lpv)
        c = lpv * (bv - mean * grstd)
        return y[...] * a + c

    ou[...] = norm(yu, 0)
    oi[...] = norm(yi, 2)


def _bn_pass(yu, yi, stats, g, b, lp):
    blk = lambda: pl.BlockSpec((_BLK, _HID), lambda i: (i, 0))
    row = lambda r: pl.BlockSpec((r, _HID), lambda i: (0, 0))
    return pl.pallas_call(
        _tc_b_body,
        grid=(_NBLK,),
        in_specs=[blk(), blk(), row(4), row(1), row(1), row(1)],
        out_specs=[blk(), blk()],
        out_shape=[jax.ShapeDtypeStruct((_N, _HID), jnp.float32),
                   jax.ShapeDtypeStruct((_N, _HID), jnp.float32)],
    )(yu, yi, stats, g, b, lp)


def kernel(x_user, x_item, edge_index_ui, edge_index_iu, LP,
           W_ui_0, W_iu_0, Wu_0, Wi_0, g_0, b_0,
           W_ui_1, W_iu_1, Wu_1, Wi_1, g_1, b_1):
    f32 = jnp.float32
    xu = x_user.astype(f32)
    xi = x_item.astype(f32)
    ei_ui = edge_index_ui.astype(jnp.int32)
    ei_iu = edge_index_iu.astype(jnp.int32)

    zeros = jnp.zeros((_RPT, _CW), f32)

    out_i, out_u, out_degi, out_degu = _sc_agg_fn()(
        xu, xi, ei_ui[0], ei_ui[1], ei_iu[0], ei_iu[1], zeros)

    aggi3 = out_i.reshape(2, _NA, _CW)
    aggu3 = out_u.reshape(2, _NA, _CW)
    degi = out_degi[:_DR].reshape(_NA)[:_N, None]
    degu = out_degu[:_DR].reshape(_NA)[:_N, None]

    yu, yi, stats = _dense_pass(xu, xi, aggu3, aggi3, degu, degi,
                                W_iu_1.astype(f32), Wu_1.astype(f32),
                                W_ui_1.astype(f32), Wi_1.astype(f32))

    lp = jnp.where(LP != 0, 1.0, 0.0).astype(f32) * jnp.ones((1, _HID), f32)
    ou, oi = _bn_pass(yu, yi, stats,
                      g_1.reshape(1, _HID).astype(f32),
                      b_1.reshape(1, _HID).astype(f32), lp)
    return ou[None, :, :], oi[None, :, :]


# speedup vs baseline: 1.0759x; 1.0099x over previous
import functools

import jax
import jax.numpy as jnp
from jax import lax
from jax.experimental import pallas as pl
from jax.experimental.pallas import tpu as pltpu
from jax.experimental.pallas import tpu_sc as plsc

_HID = 256
_N = 10000
_E = 160000
_HALF = 128
_CW = 128
_DR = 80
_NSUB = 16
_EPT = _E // _NSUB
_C = 80
_SEG = 5
_ECS = _EPT // _SEG
_NCSEG = _ECS // _C
_NA = 10240
_RPT = _NA // _NSUB
_BLK = 2000
_NBLK = _N // _BLK


@functools.cache
def _sc_agg_fn():
    mesh = plsc.VectorSubcoreMesh(core_axis_name="c", subcore_axis_name="s")
    return pl.kernel(
        _sc_body,
        out_type=[jax.ShapeDtypeStruct((2 * _NA, _CW), jnp.float32),
                  jax.ShapeDtypeStruct((2 * _NA, _CW), jnp.float32),
                  jax.ShapeDtypeStruct((2 * _DR, 128), jnp.float32),
                  jax.ShapeDtypeStruct((2 * _DR, 128), jnp.float32)],
        mesh=mesh,
        scratch_types=[
            pltpu.VMEM((_ECS,), jnp.int32),
            pltpu.VMEM((_ECS,), jnp.int32),
            pltpu.VMEM((_C,), jnp.int32),
            pltpu.VMEM((_C,), jnp.int32),
            pltpu.VMEM((_C,), jnp.int32),
            pltpu.VMEM((_C,), jnp.int32),
            pltpu.VMEM((_C,), jnp.int32),
            pltpu.VMEM((_C,), jnp.int32),
            pltpu.VMEM((_DR,), jnp.int32),
            pltpu.VMEM((_C, _CW), jnp.float32),
            pltpu.VMEM((_C, _CW), jnp.float32),
            pltpu.VMEM((_C, _CW), jnp.float32),
            pltpu.VMEM((_DR, 128), jnp.float32),
            pltpu.VMEM_SHARED((_NA, _CW), jnp.float32),
            pltpu.VMEM_SHARED((_DR, 128), jnp.float32),
            pltpu.SemaphoreType.DMA,
            pltpu.SemaphoreType.DMA,
            pltpu.SemaphoreType.DMA,
            pltpu.SemaphoreType.DMA,
            pltpu.SemaphoreType.DMA,
            pltpu.SemaphoreType.DMA,
        ],
        compiler_params=pltpu.CompilerParams(needs_layout_passes=False),
    )


def _sc_body(tu, ti, src_ui, dst_ui, src_iu, dst_iu, zeros,
             out_i, out_u, out_degi, out_degu,
             sall, dall, idx_s0, idx_d0, idx_s1, idx_d1, idx_s2, idx_d2,
             idx_io, rows0, rows1, rows2, degv, acc, deg_acc,
             gsem0, gsem1, gsem2, ssem0, ssem1, ssem2):
    cid = lax.axis_index("c")
    sid = lax.axis_index("s")
    colo = pl.multiple_of(cid * _HALF, _HALF)
    offa = cid * _NA

    for j in range(_DR // 16):
        idx_io[pl.ds(j * 16, 16)] = lax.iota(jnp.int32, 16) + (j * 16)
    ones16 = jnp.full((16,), 1.0, jnp.float32)

    def run_dir(table, src, dst, out, out_deg):
        pltpu.sync_copy(zeros, acc.at[pl.ds(sid * _RPT, _RPT)])
        pltpu.sync_copy(zeros.at[pl.ds(0, _DR)], degv)

        @pl.when(sid == 0)
        def _():
            pltpu.sync_copy(zeros.at[pl.ds(0, _DR)], deg_acc)

        plsc.subcore_barrier()

        pairs = ((idx_s0, idx_d0, rows0, gsem0, ssem0),
                 (idx_s1, idx_d1, rows1, gsem1, ssem1),
                 (idx_s2, idx_d2, rows2, gsem2, ssem2))

        def fire(kf, isb, idb, rb, gsem, ssem):
            @pl.when(kf >= 3)
            def _():
                pltpu.make_async_copy(rb, acc.at[idb], ssem).wait()

            base = kf * _C
            for j in range(_C // 16):
                sl = pl.ds(j * 16, 16)
                isb[sl] = sall[pl.ds(base + j * 16, 16)]
                idb[sl] = dall[pl.ds(base + j * 16, 16)]
            pltpu.async_copy(table.at[isb, pl.ds(colo, _HALF)], rb, gsem)

        def consume(isb, idb, rb, gsem, ssem):
            pltpu.make_async_copy(table.at[isb, pl.ds(colo, _HALF)],
                                  rb, gsem).wait()
            pltpu.async_copy(rb, acc.at[idb], ssem, add=True)
            for j in range(_C // 16):
                d16 = idb[pl.ds(j * 16, 16)]
                plsc.addupdate_scatter(
                    degv, [lax.shift_right_logical(d16, 7),
                           lax.bitwise_and(d16, 127)], ones16)

        def segment(s, carry0):
            pltpu.sync_copy(src.at[pl.ds(sid * _EPT + s * _ECS, _ECS)], sall)
            pltpu.sync_copy(dst.at[pl.ds(sid * _EPT + s * _ECS, _ECS)], dall)
            fire(0, *pairs[0])
            fire(1, *pairs[1])

            def chunk(k, carry):
                kf = k + 2
                for p in range(3):
                    @pl.when(jnp.logical_and(kf < _NCSEG,
                                             lax.rem(kf, 3) == p))
                    def _(p=p):
                        fire(kf, *pairs[p])

                for p in range(3):
                    @pl.when(lax.rem(k, 3) == p)
                    def _(p=p):
                        consume(*pairs[p])

                return carry

            lax.fori_loop(0, _NCSEG, chunk, 0)
            for p in range(3):
                isb, idb, rb, gsem, ssem = pairs[p]
                pltpu.make_async_copy(rb, acc.at[idb], ssem).wait()
            return carry0

        lax.fori_loop(0, _SEG, segment, 0)
        pltpu.sync_copy(degv, deg_acc.at[idx_io], add=True)
        plsc.subcore_barrier()
        pltpu.sync_copy(acc.at[pl.ds(sid * _RPT, _RPT)],
                        out.at[pl.ds(offa + sid * _RPT, _RPT)])

        @pl.when(sid == 0)
        def _():
            pltpu.sync_copy(deg_acc, out_deg.at[pl.ds(cid * _DR, _DR)])

        plsc.subcore_barrier()

    run_dir(tu, src_ui, dst_ui, out_i, out_degi)
    run_dir(ti, src_iu, dst_iu, out_u, out_degu)


def _tc_a_body(xu, xi, au0, au1, ai0, ai1, degu, degi, wiu, wu, wui, wi,
               yu, yi, stats):
    i = pl.program_id(0)
    aggu = jnp.concatenate([au0[0], au1[0]], axis=1)
    aggi = jnp.concatenate([ai0[0], ai1[0]], axis=1)
    du = jnp.maximum(degu[...], 1.0)
    di = jnp.maximum(degi[...], 1.0)
    mu = jnp.dot(aggu, wiu[...], preferred_element_type=jnp.float32) / du
    yu_v = jnp.maximum(
        mu + jnp.dot(xu[...], wu[...], preferred_element_type=jnp.float32), 0.0)
    mi = jnp.dot(aggi, wui[...], preferred_element_type=jnp.float32) / di
    yi_v = jnp.maximum(
        mi + jnp.dot(xi[...], wi[...], preferred_element_type=jnp.float32), 0.0)
    yu[...] = yu_v
    yi[...] = yi_v
    s = jnp.concatenate([
        jnp.sum(yu_v, axis=0, keepdims=True),
        jnp.sum(yu_v * yu_v, axis=0, keepdims=True),
        jnp.sum(yi_v, axis=0, keepdims=True),
        jnp.sum(yi_v * yi_v, axis=0, keepdims=True),
    ], axis=0)

    @pl.when(i == 0)
    def _():
        stats[...] = s

    @pl.when(i != 0)
    def _():
        stats[...] = stats[...] + s


def _dense_pass(xu, xi, aggu3, aggi3, degu, degi, wiu, wu, wui, wi):
    blk = lambda: pl.BlockSpec((_BLK, _HID), lambda i: (i, 0))
    h0 = lambda: pl.BlockSpec((1, _BLK, _HALF), lambda i: (0, i, 0))
    h1 = lambda: pl.BlockSpec((1, _BLK, _HALF), lambda i: (1, i, 0))
    wspec = lambda: pl.BlockSpec((_HID, _HID), lambda i: (0, 0))
    return pl.pallas_call(
        _tc_a_body,
        grid=(_NBLK,),
        in_specs=[blk(), blk(), h0(), h1(), h0(), h1(),
                  pl.BlockSpec((_BLK, 1), lambda i: (i, 0)),
                  pl.BlockSpec((_BLK, 1), lambda i: (i, 0)),
                  wspec(), wspec(), wspec(), wspec()],
        out_specs=[blk(), blk(),
                   pl.BlockSpec((4, _HID), lambda i: (0, 0))],
        out_shape=[jax.ShapeDtypeStruct((_N, _HID), jnp.float32),
                   jax.ShapeDtypeStruct((_N, _HID), jnp.float32),
                   jax.ShapeDtypeStruct((4, _HID), jnp.float32)],
    )(xu, xi, aggu3, aggu3, aggi3, aggi3, degu, degi, wiu, wu, wui, wi)


def _tc_b_body(yu, yi, stats, g, b, lp, ou, oi):
    n = float(_N)
    lpv = lp[...]
    gv = g[...]
    bv = b[...]

    def norm(y, r):
        mean = stats[r:r + 1, :] / n
        var = stats[r + 1:r + 2, :] / n - mean * mean
        grstd = gv * lax.rsqrt(var + 1e-5)
        a = lpv * grstd + (1.0 - lpv)
        c = lpv * (bv - mean * grstd)
        return y[...] * a + c

    ou[...] = norm(yu, 0)
    oi[...] = norm(yi, 2)


def _bn_pass(yu, yi, stats, g, b, lp):
    blk = lambda: pl.BlockSpec((_BLK, _HID), lambda i: (i, 0))
    row = lambda r: pl.BlockSpec((r, _HID), lambda i: (0, 0))
    return pl.pallas_call(
        _tc_b_body,
        grid=(_NBLK,),
        in_specs=[blk(), blk(), row(4), row(1), row(1), row(1)],
        out_specs=[blk(), blk()],
        out_shape=[jax.ShapeDtypeStruct((_N, _HID), jnp.float32),
                   jax.ShapeDtypeStruct((_N, _HID), jnp.float32)],
    )(yu, yi, stats, g, b, lp)


def kernel(x_user, x_item, edge_index_ui, edge_index_iu, LP,
           W_ui_0, W_iu_0, Wu_0, Wi_0, g_0, b_0,
           W_ui_1, W_iu_1, Wu_1, Wi_1, g_1, b_1):
    f32 = jnp.float32
    xu = x_user.astype(f32)
    xi = x_item.astype(f32)
    ei_ui = edge_index_ui.astype(jnp.int32)
    ei_iu = edge_index_iu.astype(jnp.int32)

    zeros = jnp.zeros((_RPT, _CW), f32)

    out_i, out_u, out_degi, out_degu = _sc_agg_fn()(
        xu, xi, ei_ui[0], ei_ui[1], ei_iu[0], ei_iu[1], zeros)

    aggi3 = out_i.reshape(2, _NA, _CW)
    aggu3 = out_u.reshape(2, _NA, _CW)
    degi = out_degi[:_DR].reshape(_NA)[:_N, None]
    degu = out_degu[:_DR].reshape(_NA)[:_N, None]

    yu, yi, stats = _dense_pass(xu, xi, aggu3, aggi3, degu, degi,
                                W_iu_1.astype(f32), Wu_1.astype(f32),
                                W_ui_1.astype(f32), Wi_1.astype(f32))

    lp = jnp.where(LP != 0, 1.0, 0.0).astype(f32) * jnp.ones((1, _HID), f32)
    ou, oi = _bn_pass(yu, yi, stats,
                      g_1.reshape(1, _HID).astype(f32),
                      b_1.reshape(1, _HID).astype(f32), lp)
    return ou[None, :, :], oi[None, :, :]
